# Initial kernel scaffold; baseline (speedup 1.0000x reference)
import jax
import jax.numpy as jnp
from jax.experimental import pallas as pl


def kernel(x):
    raise NotImplementedError("write your pallas kernel here")


# speedup vs baseline: 1.7798x; 1.7798x over previous
import functools

import jax
import jax.numpy as jnp
from jax.experimental import pallas as pl
from jax.experimental.pallas import tpu as pltpu

_N_REPEATS = 4
_ROWS = 16384
_COLS = 128
_BLOCK_OUT_ROWS = 1024
_THRESH = 1258292
_NAN_TOKEN = -1.0

_K0 = 0
_K1 = 42
_K2 = _K0 ^ _K1 ^ 0x1BD11BDA
_ROT_A = (13, 15, 26, 6)
_ROT_B = (17, 29, 16, 24)


def _mix(a, b, rots):
    for r in rots:
        a = a + b
        b = (b << jnp.uint32(r)) | (b >> jnp.uint32(32 - r))
        b = a ^ b
    return a, b


def _threefry_bits(idx):
    k0 = jnp.uint32(_K0)
    k1 = jnp.uint32(_K1)
    k2 = jnp.uint32(_K2)
    a = jnp.zeros_like(idx) + k0
    b = idx + k1
    a, b = _mix(a, b, _ROT_A)
    a, b = a + k1, b + (k2 + jnp.uint32(1))
    a, b = _mix(a, b, _ROT_B)
    a, b = a + k2, b + (k0 + jnp.uint32(2))
    a, b = _mix(a, b, _ROT_A)
    a, b = a + k0, b + (k1 + jnp.uint32(3))
    a, b = _mix(a, b, _ROT_B)
    a, b = a + k1, b + (k2 + jnp.uint32(4))
    a, b = _mix(a, b, _ROT_A)
    a, b = a + k2, b + (k0 + jnp.uint32(5))
    return a ^ b


def _masker_body(x_ref, x_out_ref, xv_out_ref):
    i = pl.program_id(0)
    brx = _BLOCK_OUT_ROWS // _N_REPEATS

    xb = x_ref[...]
    xrep = jnp.broadcast_to(xb[:, None, :], (brx, _N_REPEATS, _COLS))
    xrep = xrep.reshape(_BLOCK_OUT_ROWS, _COLS)

    row = jax.lax.broadcasted_iota(jnp.uint32, (_BLOCK_OUT_ROWS, _COLS), 0)
    col = jax.lax.broadcasted_iota(jnp.uint32, (_BLOCK_OUT_ROWS, _COLS), 1)
    base = jnp.uint32(i) * jnp.uint32(_BLOCK_OUT_ROWS * _COLS)
    idx = base + row * jnp.uint32(_COLS) + col

    bits = _threefry_bits(idx)
    corrupt = (bits >> jnp.uint32(9)) < jnp.uint32(_THRESH)
    corrupt = jnp.logical_and(corrupt, (row & jnp.uint32(3)) != jnp.uint32(0))

    x_out_ref[...] = xrep
    xv_out_ref[...] = jnp.where(corrupt, jnp.float32(_NAN_TOKEN), xrep)


@jax.jit
def kernel(x):
    out_rows = _ROWS * _N_REPEATS
    grid = (out_rows // _BLOCK_OUT_ROWS,)
    brx = _BLOCK_OUT_ROWS // _N_REPEATS
    X, XV = pl.pallas_call(
        _masker_body,
        grid=grid,
        in_specs=[pl.BlockSpec((brx, _COLS), lambda i: (i, 0))],
        out_specs=[
            pl.BlockSpec((_BLOCK_OUT_ROWS, _COLS), lambda i: (i, 0)),
            pl.BlockSpec((_BLOCK_OUT_ROWS, _COLS), lambda i: (i, 0)),
        ],
        out_shape=[
            jax.ShapeDtypeStruct((out_rows, _COLS), jnp.float32),
            jax.ShapeDtypeStruct((out_rows, _COLS), jnp.float32),
        ],
        compiler_params=pltpu.CompilerParams(
            dimension_semantics=("parallel",),
        ),
    )(x)
    return (X, XV)
